# baseline (speedup 1.0000x reference)
import jax
import jax.numpy as jnp
from jax.experimental import pallas as pl
from jax.experimental.pallas import tpu as pltpu

HIDDEN = 1024
FFN = 4096
NUM_EXPERTS = 8
TOP_K = 2
JITTER_EPS = 0.01
T = 2048
N = T * TOP_K
TM = 128
NT = N // TM
S = NT + NUM_EXPERTS - 1
TF = 512
F = FFN // TF
TMG = 256
TT = 256
DUMMY = N // TM


def _sparsemixer(scores, jitter_eps):
    m = jnp.max(scores, axis=-1, keepdims=True)
    sel = jnp.argmax(scores, axis=-1)
    factor = jnp.maximum(jnp.abs(scores), m)
    mask = ((m - scores) / factor) > (2.0 * jitter_eps)
    masked = jnp.where(mask, -jnp.inf, scores)
    probs = jax.nn.softmax(masked, axis=-1)
    mult = jnp.take_along_axis(probs, sel[:, None], axis=-1)[:, 0]
    return mult, sel


def _gather_kernel(ts_ref, x_ref, xs_ref):
    trow = ts_ref[...]
    cols = jax.lax.broadcasted_iota(jnp.int32, (TMG, T), 1)
    g1h = (trow == cols).astype(jnp.float32)
    xs_ref[...] = jax.lax.dot_general(
        g1h, x_ref[...], (((1,), (0,)), ((), ())),
        preferred_element_type=jnp.float32).astype(jnp.bfloat16)


def _mlp_kernel(tt_ref, te_ref, lo_ref, hi_ref, nr_ref,
                xs_ref, gs_ref, w1_ref, w3_ref, w2_ref,
                ys_ref, acc_scr, wb1_scr, wb3_scr, wb2_scr):
    f = pl.program_id(0)
    s = pl.program_id(1)

    @pl.when(nr_ref[s] == 1)
    def _convert():
        wb1_scr[...] = w1_ref[0].astype(jnp.bfloat16)
        wb3_scr[...] = w3_ref[0].astype(jnp.bfloat16)
        wb2_scr[...] = w2_ref[0].astype(jnp.bfloat16)

    xt = xs_ref[...]
    h1 = jax.lax.dot_general(xt, wb1_scr[...], (((1,), (1,)), ((), ())),
                             preferred_element_type=jnp.float32)
    h3 = jax.lax.dot_general(xt, wb3_scr[...], (((1,), (1,)), ((), ())),
                             preferred_element_type=jnp.float32)
    h = (h1 * (1.0 / (1.0 + jnp.exp(-h1))) * h3).astype(jnp.bfloat16)
    y = jax.lax.dot_general(h, wb2_scr[...], (((1,), (1,)), ((), ())),
                            preferred_element_type=jnp.float32)

    row0 = tt_ref[s] * TM
    rows = row0 + jax.lax.broadcasted_iota(jnp.int32, (TM, 1), 0)
    m = (rows >= lo_ref[s]) & (rows < hi_ref[s])
    sl = pl.ds(row0, TM)

    @pl.when(f == 0)
    def _init():
        acc_scr[sl, :] = jnp.where(m, y, acc_scr[sl, :])

    @pl.when(f > 0)
    def _acc():
        acc_scr[sl, :] = acc_scr[sl, :] + jnp.where(m, y, 0.0)

    @pl.when(f < F - 1)
    def _zero():
        ys_ref[...] = jnp.zeros((TM, HIDDEN), jnp.float32)

    @pl.when(f == F - 1)
    def _write():
        ys_ref[...] = acc_scr[sl, :] * gs_ref[...]


def _combine_kernel(ts_ref, ys_ref, out_ref):
    i = pl.program_id(0)
    rowids = jax.lax.broadcasted_iota(jnp.int32, (TT, N + TM), 0) + i * TT
    c1h = (ts_ref[...] == rowids).astype(jnp.float32)
    out_ref[...] = jax.lax.dot_general(
        c1h, ys_ref[...], (((1,), (0,)), ((), ())),
        preferred_element_type=jnp.float32)


def kernel(hidden_states, gate_w, w1, w2, w3):
    b, s, d = hidden_states.shape
    x = hidden_states.reshape(-1, d)

    router_logits = x @ gate_w.T
    mult1, sel1 = _sparsemixer(router_logits, JITTER_EPS)
    onehot1 = jax.nn.one_hot(sel1, NUM_EXPERTS, dtype=jnp.float32)
    masked_scores = jnp.where(onehot1 > 0, -jnp.inf, router_logits)
    mult2, sel2 = _sparsemixer(masked_scores, JITTER_EPS)
    sel1 = sel1.astype(jnp.int32)
    sel2 = sel2.astype(jnp.int32)

    e_all = jnp.concatenate([sel1, sel2])
    g_all = jnp.concatenate([mult1, mult2])
    t_all = jnp.concatenate([jnp.arange(T, dtype=jnp.int32)] * 2)
    perm = jnp.argsort(e_all)
    ts = t_all[perm].astype(jnp.int32)
    gs = g_all[perm]
    counts = jnp.bincount(e_all, length=NUM_EXPERTS)
    ends = jnp.cumsum(counts)
    starts = ends - counts
    tstart = jnp.arange(NT, dtype=jnp.int32) * TM
    lo = jnp.maximum(tstart[:, None], starts[None, :])
    hi = jnp.minimum(tstart[:, None] + TM, ends[None, :])
    act = hi > lo
    fidx = jnp.nonzero(act.ravel(), size=S, fill_value=-1)[0]
    vmask = fidx >= 0
    count = act.sum()
    last_fi = fidx[jnp.maximum(count - 1, 0)]
    fi = jnp.where(vmask, fidx, last_fi)
    step_tile = (fi // NUM_EXPERTS).astype(jnp.int32)
    step_e = (fi % NUM_EXPERTS).astype(jnp.int32)
    step_lo = jnp.where(vmask, lo.ravel()[fi], 0).astype(jnp.int32)
    step_hi = jnp.where(vmask, hi.ravel()[fi], 0).astype(jnp.int32)
    step_nr = jnp.concatenate(
        [jnp.ones((1,), jnp.int32),
         (step_e[1:] != step_e[:-1]).astype(jnp.int32)])

    xs = pl.pallas_call(
        _gather_kernel,
        grid=(N // TMG,),
        in_specs=[
            pl.BlockSpec((TMG, 1), lambda i: (i, 0)),
            pl.BlockSpec((T, d), lambda i: (0, 0)),
        ],
        out_specs=pl.BlockSpec((TMG, d), lambda i: (i, 0)),
        out_shape=jax.ShapeDtypeStruct((N, d), jnp.bfloat16),
    )(ts[:, None], x)

    ys = pl.pallas_call(
        _mlp_kernel,
        grid_spec=pltpu.PrefetchScalarGridSpec(
            num_scalar_prefetch=5,
            grid=(F, S),
            in_specs=[
                pl.BlockSpec((TM, d), lambda f, s, tt, te, *_: (tt[s], 0)),
                pl.BlockSpec((TM, 1), lambda f, s, tt, te, *_: (tt[s], 0)),
                pl.BlockSpec((1, TF, d), lambda f, s, tt, te, *_: (te[s], f, 0)),
                pl.BlockSpec((1, TF, d), lambda f, s, tt, te, *_: (te[s], f, 0)),
                pl.BlockSpec((1, d, TF), lambda f, s, tt, te, *_: (te[s], 0, f)),
            ],
            out_specs=pl.BlockSpec(
                (TM, d),
                lambda f, s, tt, te, *_: (jnp.where(f == F - 1, tt[s], DUMMY), 0)),
            scratch_shapes=[
                pltpu.VMEM((N, d), jnp.float32),
                pltpu.VMEM((TF, d), jnp.bfloat16),
                pltpu.VMEM((TF, d), jnp.bfloat16),
                pltpu.VMEM((d, TF), jnp.bfloat16),
            ],
        ),
        out_shape=jax.ShapeDtypeStruct((N + TM, d), jnp.float32),
    )(step_tile, step_e, step_lo, step_hi, step_nr,
      xs, gs[:, None], w1, w3, w2)

    ts_pad = jnp.concatenate(
        [ts, jnp.full((TM,), -1, jnp.int32)])[None, :]
    out = pl.pallas_call(
        _combine_kernel,
        grid=(T // TT,),
        in_specs=[
            pl.BlockSpec((1, N + TM), lambda i: (0, 0)),
            pl.BlockSpec((N + TM, d), lambda i: (0, 0)),
        ],
        out_specs=pl.BlockSpec((TT, d), lambda i: (i, 0)),
        out_shape=jax.ShapeDtypeStruct((T, d), jnp.float32),
    )(ts_pad, ys)

    return out.reshape(b, s, d), router_logits


# speedup vs baseline: 1.6140x; 1.6140x over previous
import jax
import jax.numpy as jnp
from jax.experimental import pallas as pl
from jax.experimental.pallas import tpu as pltpu

HIDDEN = 1024
FFN = 4096
NUM_EXPERTS = 8
TOP_K = 2
JITTER_EPS = 0.01
T = 2048
N = T * TOP_K
TM = 256
NT = N // TM
S = NT + NUM_EXPERTS - 1
TF = 1024
F = FFN // TF
TMG = 256
TT = 256
DUMMY = N // TM


def _sparsemixer(scores, jitter_eps):
    m = jnp.max(scores, axis=-1, keepdims=True)
    sel = jnp.argmax(scores, axis=-1)
    factor = jnp.maximum(jnp.abs(scores), m)
    mask = ((m - scores) / factor) > (2.0 * jitter_eps)
    masked = jnp.where(mask, -jnp.inf, scores)
    probs = jax.nn.softmax(masked, axis=-1)
    mult = jnp.take_along_axis(probs, sel[:, None], axis=-1)[:, 0]
    return mult, sel


def _gather_kernel(ts_ref, x_ref, xs_ref):
    trow = ts_ref[...]
    cols = jax.lax.broadcasted_iota(jnp.int32, (TMG, T), 1)
    g1h = (trow == cols).astype(jnp.float32)
    xs_ref[...] = jax.lax.dot_general(
        g1h, x_ref[...], (((1,), (0,)), ((), ())),
        preferred_element_type=jnp.float32).astype(jnp.bfloat16)


def _mlp_kernel(tt_ref, te_ref, lo_ref, hi_ref, nr_ref,
                xs_ref, gs_ref, w1_ref, w3_ref, w2_ref,
                ys_ref, acc_scr, wb1_scr, wb3_scr, wb2_scr):
    f = pl.program_id(0)
    s = pl.program_id(1)

    @pl.when(nr_ref[s] == 1)
    def _convert():
        wb1_scr[...] = w1_ref[0].astype(jnp.bfloat16)
        wb3_scr[...] = w3_ref[0].astype(jnp.bfloat16)
        wb2_scr[...] = w2_ref[0].astype(jnp.bfloat16)

    xt = xs_ref[...]
    h1 = jax.lax.dot_general(xt, wb1_scr[...], (((1,), (1,)), ((), ())),
                             preferred_element_type=jnp.float32)
    h3 = jax.lax.dot_general(xt, wb3_scr[...], (((1,), (1,)), ((), ())),
                             preferred_element_type=jnp.float32)
    h = (h1 * (1.0 / (1.0 + jnp.exp(-h1))) * h3).astype(jnp.bfloat16)
    y = jax.lax.dot_general(h, wb2_scr[...], (((1,), (1,)), ((), ())),
                            preferred_element_type=jnp.float32)

    row0 = tt_ref[s] * TM
    rows = row0 + jax.lax.broadcasted_iota(jnp.int32, (TM, 1), 0)
    m = (rows >= lo_ref[s]) & (rows < hi_ref[s])
    sl = pl.ds(row0, TM)

    @pl.when(f == 0)
    def _init():
        acc_scr[sl, :] = jnp.where(m, y, acc_scr[sl, :])

    @pl.when(f > 0)
    def _acc():
        acc_scr[sl, :] = acc_scr[sl, :] + jnp.where(m, y, 0.0)

    @pl.when(f < F - 1)
    def _zero():
        ys_ref[...] = jnp.zeros((TM, HIDDEN), jnp.float32)

    @pl.when(f == F - 1)
    def _write():
        ys_ref[...] = acc_scr[sl, :] * gs_ref[...]


def _combine_kernel(ts_ref, ys_ref, out_ref):
    i = pl.program_id(0)
    rowids = jax.lax.broadcasted_iota(jnp.int32, (TT, N + TM), 0) + i * TT
    c1h = (ts_ref[...] == rowids).astype(jnp.float32)
    out_ref[...] = jax.lax.dot_general(
        c1h, ys_ref[...], (((1,), (0,)), ((), ())),
        preferred_element_type=jnp.float32)


def kernel(hidden_states, gate_w, w1, w2, w3):
    b, s, d = hidden_states.shape
    x = hidden_states.reshape(-1, d)

    router_logits = x @ gate_w.T
    mult1, sel1 = _sparsemixer(router_logits, JITTER_EPS)
    onehot1 = jax.nn.one_hot(sel1, NUM_EXPERTS, dtype=jnp.float32)
    masked_scores = jnp.where(onehot1 > 0, -jnp.inf, router_logits)
    mult2, sel2 = _sparsemixer(masked_scores, JITTER_EPS)
    sel1 = sel1.astype(jnp.int32)
    sel2 = sel2.astype(jnp.int32)

    e_all = jnp.concatenate([sel1, sel2])
    g_all = jnp.concatenate([mult1, mult2])
    t_all = jnp.concatenate([jnp.arange(T, dtype=jnp.int32)] * 2)
    perm = jnp.argsort(e_all)
    ts = t_all[perm].astype(jnp.int32)
    gs = g_all[perm]
    counts = jnp.bincount(e_all, length=NUM_EXPERTS)
    ends = jnp.cumsum(counts)
    starts = ends - counts
    tstart = jnp.arange(NT, dtype=jnp.int32) * TM
    lo = jnp.maximum(tstart[:, None], starts[None, :])
    hi = jnp.minimum(tstart[:, None] + TM, ends[None, :])
    act = hi > lo
    fidx = jnp.nonzero(act.ravel(), size=S, fill_value=-1)[0]
    vmask = fidx >= 0
    count = act.sum()
    last_fi = fidx[jnp.maximum(count - 1, 0)]
    fi = jnp.where(vmask, fidx, last_fi)
    step_tile = (fi // NUM_EXPERTS).astype(jnp.int32)
    step_e = (fi % NUM_EXPERTS).astype(jnp.int32)
    step_lo = jnp.where(vmask, lo.ravel()[fi], 0).astype(jnp.int32)
    step_hi = jnp.where(vmask, hi.ravel()[fi], 0).astype(jnp.int32)
    step_nr = jnp.concatenate(
        [jnp.ones((1,), jnp.int32),
         (step_e[1:] != step_e[:-1]).astype(jnp.int32)])

    xs = pl.pallas_call(
        _gather_kernel,
        grid=(N // TMG,),
        in_specs=[
            pl.BlockSpec((TMG, 1), lambda i: (i, 0)),
            pl.BlockSpec((T, d), lambda i: (0, 0)),
        ],
        out_specs=pl.BlockSpec((TMG, d), lambda i: (i, 0)),
        out_shape=jax.ShapeDtypeStruct((N, d), jnp.bfloat16),
    )(ts[:, None], x)

    ys = pl.pallas_call(
        _mlp_kernel,
        grid_spec=pltpu.PrefetchScalarGridSpec(
            num_scalar_prefetch=5,
            grid=(F, S),
            in_specs=[
                pl.BlockSpec((TM, d), lambda f, s, tt, te, *_: (tt[s], 0)),
                pl.BlockSpec((TM, 1), lambda f, s, tt, te, *_: (tt[s], 0)),
                pl.BlockSpec((1, TF, d), lambda f, s, tt, te, *_: (te[s], f, 0)),
                pl.BlockSpec((1, TF, d), lambda f, s, tt, te, *_: (te[s], f, 0)),
                pl.BlockSpec((1, d, TF), lambda f, s, tt, te, *_: (te[s], 0, f)),
            ],
            out_specs=pl.BlockSpec(
                (TM, d),
                lambda f, s, tt, te, *_: (jnp.where(f == F - 1, tt[s], DUMMY), 0)),
            scratch_shapes=[
                pltpu.VMEM((N, d), jnp.float32),
                pltpu.VMEM((TF, d), jnp.bfloat16),
                pltpu.VMEM((TF, d), jnp.bfloat16),
                pltpu.VMEM((d, TF), jnp.bfloat16),
            ],
        ),
        out_shape=jax.ShapeDtypeStruct((N + TM, d), jnp.float32),
    )(step_tile, step_e, step_lo, step_hi, step_nr,
      xs, gs[:, None], w1, w3, w2)

    ts_pad = jnp.concatenate(
        [ts, jnp.full((TM,), -1, jnp.int32)])[None, :]
    out = pl.pallas_call(
        _combine_kernel,
        grid=(T // TT,),
        in_specs=[
            pl.BlockSpec((1, N + TM), lambda i: (0, 0)),
            pl.BlockSpec((N + TM, d), lambda i: (0, 0)),
        ],
        out_specs=pl.BlockSpec((TT, d), lambda i: (i, 0)),
        out_shape=jax.ShapeDtypeStruct((T, d), jnp.float32),
    )(ts_pad, ys)

    return out.reshape(b, s, d), router_logits


# speedup vs baseline: 1.6887x; 1.0463x over previous
import jax
import jax.numpy as jnp
from jax.experimental import pallas as pl
from jax.experimental.pallas import tpu as pltpu

HIDDEN = 1024
FFN = 4096
NUM_EXPERTS = 8
TOP_K = 2
JITTER_EPS = 0.01
T = 2048
N = T * TOP_K
TM = 512
NT = N // TM
S = NT + NUM_EXPERTS - 1
TF = 1024
F = FFN // TF
TMG = 256
TT = 256
DUMMY = N // TM


def _sparsemixer(scores, jitter_eps):
    m = jnp.max(scores, axis=-1, keepdims=True)
    sel = jnp.argmax(scores, axis=-1)
    factor = jnp.maximum(jnp.abs(scores), m)
    mask = ((m - scores) / factor) > (2.0 * jitter_eps)
    masked = jnp.where(mask, -jnp.inf, scores)
    probs = jax.nn.softmax(masked, axis=-1)
    mult = jnp.take_along_axis(probs, sel[:, None], axis=-1)[:, 0]
    return mult, sel


def _gather_kernel(ts_ref, x_ref, xs_ref):
    trow = ts_ref[...]
    cols = jax.lax.broadcasted_iota(jnp.int32, (TMG, T), 1)
    g1h = (trow == cols).astype(jnp.float32)
    xs_ref[...] = jax.lax.dot_general(
        g1h, x_ref[...], (((1,), (0,)), ((), ())),
        preferred_element_type=jnp.float32).astype(jnp.bfloat16)


def _mlp_kernel(tt_ref, te_ref, lo_ref, hi_ref, nr_ref,
                xs_ref, gs_ref, w1_ref, w3_ref, w2_ref,
                ys_ref, acc_scr, wb1_scr, wb3_scr, wb2_scr):
    f = pl.program_id(0)
    s = pl.program_id(1)

    @pl.when(nr_ref[s] == 1)
    def _convert():
        wb1_scr[...] = w1_ref[0].astype(jnp.bfloat16)
        wb3_scr[...] = w3_ref[0].astype(jnp.bfloat16)
        wb2_scr[...] = w2_ref[0].astype(jnp.bfloat16)

    xt = xs_ref[...]
    h1 = jax.lax.dot_general(xt, wb1_scr[...], (((1,), (1,)), ((), ())),
                             preferred_element_type=jnp.float32)
    h3 = jax.lax.dot_general(xt, wb3_scr[...], (((1,), (1,)), ((), ())),
                             preferred_element_type=jnp.float32)
    h = (h1 * (1.0 / (1.0 + jnp.exp(-h1))) * h3).astype(jnp.bfloat16)
    y = jax.lax.dot_general(h, wb2_scr[...], (((1,), (1,)), ((), ())),
                            preferred_element_type=jnp.float32)

    row0 = tt_ref[s] * TM
    rows = row0 + jax.lax.broadcasted_iota(jnp.int32, (TM, 1), 0)
    m = (rows >= lo_ref[s]) & (rows < hi_ref[s])
    sl = pl.ds(row0, TM)

    @pl.when(f == 0)
    def _init():
        acc_scr[sl, :] = jnp.where(m, y, acc_scr[sl, :])

    @pl.when(f > 0)
    def _acc():
        acc_scr[sl, :] = acc_scr[sl, :] + jnp.where(m, y, 0.0)

    @pl.when(f < F - 1)
    def _zero():
        ys_ref[...] = jnp.zeros((TM, HIDDEN), jnp.float32)

    @pl.when(f == F - 1)
    def _write():
        ys_ref[...] = acc_scr[sl, :] * gs_ref[...]


def _combine_kernel(ts_ref, ys_ref, out_ref):
    i = pl.program_id(0)
    rowids = jax.lax.broadcasted_iota(jnp.int32, (TT, N + TM), 0) + i * TT
    c1h = (ts_ref[...] == rowids).astype(jnp.float32)
    out_ref[...] = jax.lax.dot_general(
        c1h, ys_ref[...], (((1,), (0,)), ((), ())),
        preferred_element_type=jnp.float32)


def kernel(hidden_states, gate_w, w1, w2, w3):
    b, s, d = hidden_states.shape
    x = hidden_states.reshape(-1, d)

    router_logits = x @ gate_w.T
    mult1, sel1 = _sparsemixer(router_logits, JITTER_EPS)
    onehot1 = jax.nn.one_hot(sel1, NUM_EXPERTS, dtype=jnp.float32)
    masked_scores = jnp.where(onehot1 > 0, -jnp.inf, router_logits)
    mult2, sel2 = _sparsemixer(masked_scores, JITTER_EPS)
    sel1 = sel1.astype(jnp.int32)
    sel2 = sel2.astype(jnp.int32)

    e_all = jnp.concatenate([sel1, sel2])
    g_all = jnp.concatenate([mult1, mult2])
    t_all = jnp.concatenate([jnp.arange(T, dtype=jnp.int32)] * 2)
    perm = jnp.argsort(e_all)
    ts = t_all[perm].astype(jnp.int32)
    gs = g_all[perm]
    counts = jnp.bincount(e_all, length=NUM_EXPERTS)
    ends = jnp.cumsum(counts)
    starts = ends - counts
    tstart = jnp.arange(NT, dtype=jnp.int32) * TM
    lo = jnp.maximum(tstart[:, None], starts[None, :])
    hi = jnp.minimum(tstart[:, None] + TM, ends[None, :])
    act = hi > lo
    fidx = jnp.nonzero(act.ravel(), size=S, fill_value=-1)[0]
    vmask = fidx >= 0
    count = act.sum()
    last_fi = fidx[jnp.maximum(count - 1, 0)]
    fi = jnp.where(vmask, fidx, last_fi)
    step_tile = (fi // NUM_EXPERTS).astype(jnp.int32)
    step_e = (fi % NUM_EXPERTS).astype(jnp.int32)
    step_lo = jnp.where(vmask, lo.ravel()[fi], 0).astype(jnp.int32)
    step_hi = jnp.where(vmask, hi.ravel()[fi], 0).astype(jnp.int32)
    step_nr = jnp.concatenate(
        [jnp.ones((1,), jnp.int32),
         (step_e[1:] != step_e[:-1]).astype(jnp.int32)])

    xs = pl.pallas_call(
        _gather_kernel,
        grid=(N // TMG,),
        in_specs=[
            pl.BlockSpec((TMG, 1), lambda i: (i, 0)),
            pl.BlockSpec((T, d), lambda i: (0, 0)),
        ],
        out_specs=pl.BlockSpec((TMG, d), lambda i: (i, 0)),
        out_shape=jax.ShapeDtypeStruct((N, d), jnp.bfloat16),
    )(ts[:, None], x)

    ys = pl.pallas_call(
        _mlp_kernel,
        grid_spec=pltpu.PrefetchScalarGridSpec(
            num_scalar_prefetch=5,
            grid=(F, S),
            in_specs=[
                pl.BlockSpec((TM, d), lambda f, s, tt, te, *_: (tt[s], 0)),
                pl.BlockSpec((TM, 1), lambda f, s, tt, te, *_: (tt[s], 0)),
                pl.BlockSpec((1, TF, d), lambda f, s, tt, te, *_: (te[s], f, 0)),
                pl.BlockSpec((1, TF, d), lambda f, s, tt, te, *_: (te[s], f, 0)),
                pl.BlockSpec((1, d, TF), lambda f, s, tt, te, *_: (te[s], 0, f)),
            ],
            out_specs=pl.BlockSpec(
                (TM, d),
                lambda f, s, tt, te, *_: (jnp.where(f == F - 1, tt[s], DUMMY), 0)),
            scratch_shapes=[
                pltpu.VMEM((N, d), jnp.float32),
                pltpu.VMEM((TF, d), jnp.bfloat16),
                pltpu.VMEM((TF, d), jnp.bfloat16),
                pltpu.VMEM((d, TF), jnp.bfloat16),
            ],
        ),
        out_shape=jax.ShapeDtypeStruct((N + TM, d), jnp.float32),
    )(step_tile, step_e, step_lo, step_hi, step_nr,
      xs, gs[:, None], w1, w3, w2)

    ts_pad = jnp.concatenate(
        [ts, jnp.full((TM,), -1, jnp.int32)])[None, :]
    out = pl.pallas_call(
        _combine_kernel,
        grid=(T // TT,),
        in_specs=[
            pl.BlockSpec((1, N + TM), lambda i: (0, 0)),
            pl.BlockSpec((N + TM, d), lambda i: (0, 0)),
        ],
        out_specs=pl.BlockSpec((TT, d), lambda i: (i, 0)),
        out_shape=jax.ShapeDtypeStruct((T, d), jnp.float32),
    )(ts_pad, ys)

    return out.reshape(b, s, d), router_logits


# speedup vs baseline: 1.6989x; 1.0061x over previous
import jax
import jax.numpy as jnp
from jax.experimental import pallas as pl
from jax.experimental.pallas import tpu as pltpu

HIDDEN = 1024
FFN = 4096
NUM_EXPERTS = 8
TOP_K = 2
JITTER_EPS = 0.01
T = 2048
N = T * TOP_K
TM = 512
NT = N // TM
S = NT + NUM_EXPERTS - 1
TF = 1024
F = FFN // TF
TMG = 256
TT = 256
DUMMY = N // TM


def _sparsemixer(scores, jitter_eps):
    m = jnp.max(scores, axis=-1, keepdims=True)
    sel = jnp.argmax(scores, axis=-1)
    factor = jnp.maximum(jnp.abs(scores), m)
    mask = ((m - scores) / factor) > (2.0 * jitter_eps)
    masked = jnp.where(mask, -jnp.inf, scores)
    probs = jax.nn.softmax(masked, axis=-1)
    mult = jnp.take_along_axis(probs, sel[:, None], axis=-1)[:, 0]
    return mult, sel


def _gather_kernel(ts_ref, x_ref, xs_ref):
    trow = ts_ref[...]
    cols = jax.lax.broadcasted_iota(jnp.int32, (TMG, T), 1)
    g1h = (trow == cols).astype(jnp.float32)
    xs_ref[...] = jax.lax.dot_general(
        g1h, x_ref[...], (((1,), (0,)), ((), ())),
        preferred_element_type=jnp.float32).astype(jnp.bfloat16)


def _mlp_kernel(tt_ref, te_ref, lo_ref, hi_ref, nr_ref,
                xs_ref, gs_ref, w1_ref, w3_ref, w2_ref,
                ys_ref, acc_scr, wb1_scr, wb3_scr, wb2_scr):
    f = pl.program_id(0)
    s = pl.program_id(1)

    @pl.when(nr_ref[s] == 1)
    def _convert():
        wb1_scr[...] = w1_ref[0].astype(jnp.bfloat16)
        wb3_scr[...] = w3_ref[0].astype(jnp.bfloat16)
        wb2_scr[...] = w2_ref[0].astype(jnp.bfloat16)

    xt = xs_ref[...]
    h1 = jax.lax.dot_general(xt, wb1_scr[...], (((1,), (1,)), ((), ())),
                             preferred_element_type=jnp.float32)
    h3 = jax.lax.dot_general(xt, wb3_scr[...], (((1,), (1,)), ((), ())),
                             preferred_element_type=jnp.float32)
    h = (h1 * (1.0 / (1.0 + jnp.exp(-h1))) * h3).astype(jnp.bfloat16)
    y = jax.lax.dot_general(h, wb2_scr[...], (((1,), (1,)), ((), ())),
                            preferred_element_type=jnp.float32)

    row0 = tt_ref[s] * TM
    rows = row0 + jax.lax.broadcasted_iota(jnp.int32, (TM, 1), 0)
    m = (rows >= lo_ref[s]) & (rows < hi_ref[s])
    sl = pl.ds(row0, TM)

    @pl.when(f == 0)
    def _init():
        acc_scr[sl, :] = jnp.where(m, y, acc_scr[sl, :])

    @pl.when(f > 0)
    def _acc():
        acc_scr[sl, :] = acc_scr[sl, :] + jnp.where(m, y, 0.0)

    @pl.when((f == 0) & (s == 0))
    def _zero():
        ys_ref[...] = jnp.zeros((TM, HIDDEN), jnp.float32)

    @pl.when(f == F - 1)
    def _write():
        ys_ref[...] = acc_scr[sl, :] * gs_ref[...]


def _combine_kernel(ts_ref, ys_ref, out_ref):
    i = pl.program_id(0)
    rowids = jax.lax.broadcasted_iota(jnp.int32, (TT, N + TM), 0) + i * TT
    c1h = (ts_ref[...] == rowids).astype(jnp.float32)
    out_ref[...] = jax.lax.dot_general(
        c1h, ys_ref[...], (((1,), (0,)), ((), ())),
        preferred_element_type=jnp.float32)


def kernel(hidden_states, gate_w, w1, w2, w3):
    b, s, d = hidden_states.shape
    x = hidden_states.reshape(-1, d)

    router_logits = x @ gate_w.T
    mult1, sel1 = _sparsemixer(router_logits, JITTER_EPS)
    onehot1 = jax.nn.one_hot(sel1, NUM_EXPERTS, dtype=jnp.float32)
    masked_scores = jnp.where(onehot1 > 0, -jnp.inf, router_logits)
    mult2, sel2 = _sparsemixer(masked_scores, JITTER_EPS)
    sel1 = sel1.astype(jnp.int32)
    sel2 = sel2.astype(jnp.int32)

    e_all = jnp.concatenate([sel1, sel2])
    g_all = jnp.concatenate([mult1, mult2])
    t_all = jnp.concatenate([jnp.arange(T, dtype=jnp.int32)] * 2)
    perm = jnp.argsort(e_all)
    ts = t_all[perm].astype(jnp.int32)
    gs = g_all[perm]
    counts = jnp.bincount(e_all, length=NUM_EXPERTS)
    ends = jnp.cumsum(counts)
    starts = ends - counts
    tstart = jnp.arange(NT, dtype=jnp.int32) * TM
    lo = jnp.maximum(tstart[:, None], starts[None, :])
    hi = jnp.minimum(tstart[:, None] + TM, ends[None, :])
    act = hi > lo
    fidx = jnp.nonzero(act.ravel(), size=S, fill_value=-1)[0]
    vmask = fidx >= 0
    count = act.sum()
    last_fi = fidx[jnp.maximum(count - 1, 0)]
    fi = jnp.where(vmask, fidx, last_fi)
    step_tile = (fi // NUM_EXPERTS).astype(jnp.int32)
    step_e = (fi % NUM_EXPERTS).astype(jnp.int32)
    step_lo = jnp.where(vmask, lo.ravel()[fi], 0).astype(jnp.int32)
    step_hi = jnp.where(vmask, hi.ravel()[fi], 0).astype(jnp.int32)
    step_nr = jnp.concatenate(
        [jnp.ones((1,), jnp.int32),
         (step_e[1:] != step_e[:-1]).astype(jnp.int32)])

    xs = pl.pallas_call(
        _gather_kernel,
        grid=(N // TMG,),
        in_specs=[
            pl.BlockSpec((TMG, 1), lambda i: (i, 0)),
            pl.BlockSpec((T, d), lambda i: (0, 0)),
        ],
        out_specs=pl.BlockSpec((TMG, d), lambda i: (i, 0)),
        out_shape=jax.ShapeDtypeStruct((N, d), jnp.bfloat16),
    )(ts[:, None], x)

    ys = pl.pallas_call(
        _mlp_kernel,
        grid_spec=pltpu.PrefetchScalarGridSpec(
            num_scalar_prefetch=5,
            grid=(F, S),
            in_specs=[
                pl.BlockSpec((TM, d), lambda f, s, tt, te, *_: (tt[s], 0)),
                pl.BlockSpec((TM, 1), lambda f, s, tt, te, *_: (tt[s], 0)),
                pl.BlockSpec((1, TF, d), lambda f, s, tt, te, *_: (te[s], f, 0)),
                pl.BlockSpec((1, TF, d), lambda f, s, tt, te, *_: (te[s], f, 0)),
                pl.BlockSpec((1, d, TF), lambda f, s, tt, te, *_: (te[s], 0, f)),
            ],
            out_specs=pl.BlockSpec(
                (TM, d),
                lambda f, s, tt, te, *_: (jnp.where(f == F - 1, tt[s], DUMMY), 0)),
            scratch_shapes=[
                pltpu.VMEM((N, d), jnp.float32),
                pltpu.VMEM((TF, d), jnp.bfloat16),
                pltpu.VMEM((TF, d), jnp.bfloat16),
                pltpu.VMEM((d, TF), jnp.bfloat16),
            ],
        ),
        out_shape=jax.ShapeDtypeStruct((N + TM, d), jnp.float32),
    )(step_tile, step_e, step_lo, step_hi, step_nr,
      xs, gs[:, None], w1, w3, w2)

    ts_pad = jnp.concatenate(
        [ts, jnp.full((TM,), -1, jnp.int32)])[None, :]
    out = pl.pallas_call(
        _combine_kernel,
        grid=(T // TT,),
        in_specs=[
            pl.BlockSpec((1, N + TM), lambda i: (0, 0)),
            pl.BlockSpec((N + TM, d), lambda i: (0, 0)),
        ],
        out_specs=pl.BlockSpec((TT, d), lambda i: (i, 0)),
        out_shape=jax.ShapeDtypeStruct((T, d), jnp.float32),
    )(ts_pad, ys)

    return out.reshape(b, s, d), router_logits
